# Initial kernel scaffold; baseline (speedup 1.0000x reference)
import jax
import jax.numpy as jnp
from jax.experimental import pallas as pl


def kernel(X, edge_index, batch, bn_gamma, bn_beta, W1a, b1a, W1b, b1b, W1c, b1c, Wc1, bc1, cn_gamma, cn_beta, Wc2, bc2, t, W2a, b2a, W2b, b2b, W2c, b2c, W2d, b2d, Wo, bo):
    raise NotImplementedError("write your pallas kernel here")


# speedup vs baseline: 5.6225x; 5.6225x over previous
import functools

import jax
import jax.numpy as jnp
from jax.experimental import pallas as pl

_N = 10000
_G = 64
_D = 48


def _fused_body(x_ref, b_ref, g_ref, be_ref, w2a_ref, b2a_ref, w2b_ref, b2b_ref,
                w2c_ref, b2c_ref, w2d_ref, b2d_ref, wo_ref, bo_ref, out_ref):
    x = x_ref[:, :]
    n = jnp.float32(_N)

    colsum = jnp.sum(x, axis=0, keepdims=True)
    colsq = jnp.sum(x * x, axis=0, keepdims=True)
    mu = colsum / n
    var = colsq / n - mu * mu
    inv = jax.lax.rsqrt(var + 1e-5)

    gid = b_ref[:, :]
    onehot = (gid == jax.lax.broadcasted_iota(jnp.int32, (1, _G), 1)
              ).astype(jnp.float32)
    dn = (((0,), (0,)), ((), ()))
    sums = jax.lax.dot_general(onehot, x, dn,
                               preferred_element_type=jnp.float32)
    ones = jnp.ones((_N, 1), jnp.float32)
    cnt = jax.lax.dot_general(onehot, ones, dn,
                              preferred_element_type=jnp.float32)

    gamma = g_ref[:, :]
    beta = be_ref[:, :]
    seg_bn = (sums - cnt * mu) * inv * gamma + cnt * beta
    x3 = seg_bn / jnp.maximum(cnt, 1.0)

    def mm(a, w_ref, bias_ref):
        return jax.lax.dot_general(
            a, w_ref[:, :], (((1,), (0,)), ((), ())),
            preferred_element_type=jnp.float32) + bias_ref[:, :]

    h = jnp.maximum(mm(x3, w2a_ref, b2a_ref), 0.0)
    h = jnp.maximum(mm(h, w2b_ref, b2b_ref), 0.0)
    h = jnp.maximum(mm(h, w2c_ref, b2c_ref), 0.0)
    h = mm(h, w2d_ref, b2d_ref)
    out_ref[:, :] = mm(h, wo_ref, bo_ref)


@functools.partial(jax.jit, static_argnames=())
def _fused(X, batch2d, bn_gamma, bn_beta, W2a, b2a, W2b, b2b, W2c, b2c,
           W2d, b2d, Wo, bo):
    return pl.pallas_call(
        _fused_body,
        out_shape=jax.ShapeDtypeStruct((_G, 1), jnp.float32),
    )(X, batch2d, bn_gamma, bn_beta, W2a, b2a, W2b, b2b, W2c, b2c,
      W2d, b2d, Wo, bo)


def kernel(X, edge_index, batch, bn_gamma, bn_beta, W1a, b1a, W1b, b1b,
           W1c, b1c, Wc1, bc1, cn_gamma, cn_beta, Wc2, bc2, t,
           W2a, b2a, W2b, b2b, W2c, b2c, W2d, b2d, Wo, bo):
    return _fused(
        X,
        batch.reshape(_N, 1),
        bn_gamma.reshape(1, _D),
        bn_beta.reshape(1, _D),
        W2a, b2a.reshape(1, -1),
        W2b, b2b.reshape(1, -1),
        W2c, b2c.reshape(1, -1),
        W2d, b2d.reshape(1, -1),
        Wo, bo.reshape(1, -1),
    )


# speedup vs baseline: 8.3400x; 1.4833x over previous
import jax
import jax.numpy as jnp
from jax.experimental import pallas as pl

_N = 10000
_G = 64
_D = 48


def _fused_body(x_ref, b_ref, g_ref, be_ref, w2a_ref, b2a_ref, w2b_ref, b2b_ref,
                w2c_ref, b2c_ref, w2d_ref, b2d_ref, wo_ref, bo_ref, out_ref):
    x = x_ref[:, :]
    n = jnp.float32(_N)

    colsum = jnp.sum(x, axis=0, keepdims=True)
    colsq = jnp.sum(x * x, axis=0, keepdims=True)
    mu = colsum / n
    var = colsq / n - mu * mu
    inv = jax.lax.rsqrt(var + 1e-5)

    bat = b_ref[:, :]
    onehot_t = (bat == jax.lax.broadcasted_iota(jnp.int32, (_G, 1), 0)
                ).astype(jnp.float32)
    sums = jax.lax.dot_general(onehot_t, x, (((1,), (0,)), ((), ())),
                               preferred_element_type=jnp.float32)
    cnt = jnp.sum(onehot_t, axis=1, keepdims=True)

    gamma = g_ref[:]
    beta = be_ref[:]
    seg_bn = (sums - cnt * mu) * inv * gamma + cnt * beta
    x3 = seg_bn / jnp.maximum(cnt, 1.0)

    def mm(a, w_ref, bias_ref):
        return jax.lax.dot_general(
            a, w_ref[:, :], (((1,), (0,)), ((), ())),
            preferred_element_type=jnp.float32) + bias_ref[:]

    h = jnp.maximum(mm(x3, w2a_ref, b2a_ref), 0.0)
    h = jnp.maximum(mm(h, w2b_ref, b2b_ref), 0.0)
    h = jnp.maximum(mm(h, w2c_ref, b2c_ref), 0.0)
    h = mm(h, w2d_ref, b2d_ref)
    out_ref[:, :] = mm(h, wo_ref, bo_ref)


@jax.jit
def _fused(X, batch_row, bn_gamma, bn_beta, W2a, b2a, W2b, b2b, W2c, b2c,
           W2d, b2d, Wo, bo):
    return pl.pallas_call(
        _fused_body,
        out_shape=jax.ShapeDtypeStruct((_G, 1), jnp.float32),
    )(X, batch_row, bn_gamma, bn_beta, W2a, b2a, W2b, b2b, W2c, b2c,
      W2d, b2d, Wo, bo)


def kernel(X, edge_index, batch, bn_gamma, bn_beta, W1a, b1a, W1b, b1b,
           W1c, b1c, Wc1, bc1, cn_gamma, cn_beta, Wc2, bc2, t,
           W2a, b2a, W2b, b2b, W2c, b2c, W2d, b2d, Wo, bo):
    return _fused(
        X,
        batch.reshape(1, _N),
        bn_gamma, bn_beta,
        W2a, b2a, W2b, b2b, W2c, b2c, W2d, b2d, Wo, bo,
    )
